# baseline (speedup 1.0000x reference)
import functools

import jax
import jax.numpy as jnp
from jax.experimental import pallas as pl
from jax.experimental.pallas import tpu as pltpu

_B = 1024
_D = 128
_M = 16384
_BLK = 2048
_NBLK = _M // _BLK


def _mm_stage(x_ref, lab_ref, mem_ref, lmem_ref, xa_ref, buf_ref, lye_ref,
              substitute_batch):
    if substitute_batch:
        y = jnp.concatenate(
            [x_ref[...], mem_ref[pl.ds(_B, _BLK - _B), :]], axis=0)
        lye_ref[...] = jnp.concatenate(
            [lab_ref[...].reshape(1, _B),
             lmem_ref[pl.ds(_B, _BLK - _B)].reshape(1, _BLK - _B)], axis=1)
    else:
        y = mem_ref[...]
        lye_ref[...] = lmem_ref[...].reshape(1, _BLK)
    ysq = jnp.sum(y * y, axis=1, keepdims=True)
    ya = jnp.concatenate(
        [y, jnp.ones((_BLK, 1), jnp.float32), ysq], axis=1)
    buf_ref[...] = jax.lax.dot_general(
        xa_ref[...], ya, (((1,), (1,)), ((), ())),
        preferred_element_type=jnp.float32)


def _reduce_stage(j, buf_ref, lye_ref, lcol_ref, acc_ref):
    d2 = buf_ref[...]
    m = jnp.maximum(d2, 1e-12)
    d = m * jax.lax.rsqrt(m)

    one = jnp.float32(1.0)
    zero = jnp.float32(0.0)
    mf = jnp.where(lcol_ref[...] == lye_ref[...], one, zero)
    ltf = jnp.where(d2 < one, one, zero)
    nf = ltf * (one - mf)

    acc_ref[0] = acc_ref[0] + jnp.sum(d * mf)
    acc_ref[1] = acc_ref[1] + jnp.sum(mf)
    acc_ref[2] = acc_ref[2] + jnp.sum(d * nf)
    acc_ref[3] = acc_ref[3] + jnp.sum(nf)

    @pl.when(j == 1)
    def _self_correction():
        rows = jax.lax.broadcasted_iota(jnp.int32, (_B, _BLK), 0)
        cols = jax.lax.broadcasted_iota(jnp.int32, (_B, _BLK), 1)
        diag_sum = jnp.sum(jnp.where(rows == cols, d, zero))
        acc_ref[0] = acc_ref[0] - diag_sum
        acc_ref[1] = acc_ref[1] - jnp.float32(_B)


def _loss_block(x_ref, lab_ref, mem_ref, lmem_ref, out_ref,
                xa_ref, lcol_ref, buf_a, buf_b, lye_a, lye_b, acc_ref):
    j = pl.program_id(0)

    @pl.when(j == 0)
    def _first():
        acc_ref[0] = 0.0
        acc_ref[1] = 0.0
        acc_ref[2] = 0.0
        acc_ref[3] = 0.0
        x = x_ref[...]
        xsq = jnp.sum(x * x, axis=1, keepdims=True)
        xa_ref[...] = jnp.concatenate(
            [x * jnp.float32(-2.0), xsq, jnp.ones((_B, 1), jnp.float32)],
            axis=1)
        lcol_ref[...] = lab_ref[...].reshape(1, _B).T
        _mm_stage(x_ref, lab_ref, mem_ref, lmem_ref, xa_ref, buf_a, lye_a,
                  substitute_batch=True)

    @pl.when(jnp.logical_and(j > 0, j % 2 == 1))
    def _odd():
        _mm_stage(x_ref, lab_ref, mem_ref, lmem_ref, xa_ref, buf_b, lye_b,
                  substitute_batch=False)
        _reduce_stage(j, buf_a, lye_a, lcol_ref, acc_ref)

    @pl.when(jnp.logical_and(j > 0, j % 2 == 0))
    def _even():
        _mm_stage(x_ref, lab_ref, mem_ref, lmem_ref, xa_ref, buf_a, lye_a,
                  substitute_batch=False)
        _reduce_stage(j, buf_b, lye_b, lcol_ref, acc_ref)

    @pl.when(j == _NBLK)
    def _finish():
        pos_loss = acc_ref[0] / jnp.maximum(acc_ref[1], 1.0)
        neg_loss = (acc_ref[3] - acc_ref[2]) / jnp.maximum(acc_ref[3], 1.0)
        out_ref[0] = pos_loss + neg_loss


@functools.partial(jax.jit, static_argnames=())
def kernel(embeddings, labels, embedding_memory, label_memory):
    out = pl.pallas_call(
        _loss_block,
        grid=(_NBLK + 1,),
        in_specs=[
            pl.BlockSpec((_B, _D), lambda j: (0, 0)),
            pl.BlockSpec((_B,), lambda j: (0,)),
            pl.BlockSpec((_BLK, _D), lambda j: (jnp.minimum(j, _NBLK - 1), 0)),
            pl.BlockSpec((_BLK,), lambda j: (jnp.minimum(j, _NBLK - 1),)),
        ],
        out_specs=pl.BlockSpec(memory_space=pltpu.SMEM),
        out_shape=jax.ShapeDtypeStruct((1,), jnp.float32),
        scratch_shapes=[
            pltpu.VMEM((_B, _D + 2), jnp.float32),
            pltpu.VMEM((_B, 1), jnp.int32),
            pltpu.VMEM((_B, _BLK), jnp.float32),
            pltpu.VMEM((_B, _BLK), jnp.float32),
            pltpu.VMEM((1, _BLK), jnp.int32),
            pltpu.VMEM((1, _BLK), jnp.int32),
            pltpu.SMEM((4,), jnp.float32),
        ],
    )(embeddings, labels, embedding_memory, label_memory)
    return out[0]


# speedup vs baseline: 1.0334x; 1.0334x over previous
import functools

import jax
import jax.numpy as jnp
from jax.experimental import pallas as pl
from jax.experimental.pallas import tpu as pltpu

_B = 1024
_D = 128
_M = 16384
_BLK = 2048
_NBLK = _M // _BLK


def _loss_block(x_ref, lab_ref, mem_ref, lmem_ref, out_ref,
                xa_ref, lcol_ref, acc_ref):
    j = pl.program_id(0)
    is_batch = j == 0

    @pl.when(is_batch)
    def _init():
        acc_ref[0] = 0.0
        acc_ref[1] = 0.0
        acc_ref[2] = 0.0
        acc_ref[3] = 0.0
        x = x_ref[...]
        xsq = jnp.sum(x * x, axis=1, keepdims=True)
        xa_ref[...] = jnp.concatenate(
            [x * jnp.float32(-2.0), xsq, jnp.ones((_B, 1), jnp.float32)],
            axis=1)
        lcol_ref[...] = lab_ref[...].reshape(1, _B).T

    y0 = jnp.concatenate(
        [x_ref[...], mem_ref[pl.ds(_B, _BLK - _B), :]], axis=0)
    ly0 = jnp.concatenate(
        [lab_ref[...].reshape(1, _B),
         lmem_ref[pl.ds(_B, _BLK - _B)].reshape(1, _BLK - _B)], axis=1)
    y = jnp.where(is_batch, y0, mem_ref[...])
    ly = jnp.where(is_batch, ly0, lmem_ref[...].reshape(1, _BLK))

    ysq = jnp.sum(y * y, axis=1, keepdims=True)
    ya = jnp.concatenate(
        [y, jnp.ones((_BLK, 1), jnp.float32), ysq], axis=1)
    d2 = jax.lax.dot_general(
        xa_ref[...], ya, (((1,), (1,)), ((), ())),
        preferred_element_type=jnp.float32)
    m = jnp.maximum(d2, 1e-12)
    d = m * jax.lax.rsqrt(m)

    one = jnp.float32(1.0)
    zero = jnp.float32(0.0)
    mf = jnp.where(lcol_ref[...] == ly, one, zero)
    ltf = jnp.where(d2 < one, one, zero)
    nf = ltf * (one - mf)

    acc_ref[0] = acc_ref[0] + jnp.sum(d * mf)
    acc_ref[1] = acc_ref[1] + jnp.sum(mf)
    acc_ref[2] = acc_ref[2] + jnp.sum(d * nf)
    acc_ref[3] = acc_ref[3] + jnp.sum(nf)

    @pl.when(is_batch)
    def _self_correction():
        rows = jax.lax.broadcasted_iota(jnp.int32, (_B, _BLK), 0)
        cols = jax.lax.broadcasted_iota(jnp.int32, (_B, _BLK), 1)
        diag_sum = jnp.sum(jnp.where(rows == cols, d, zero))
        acc_ref[0] = acc_ref[0] - diag_sum
        acc_ref[1] = acc_ref[1] - jnp.float32(_B)

    @pl.when(j == _NBLK - 1)
    def _finish():
        pos_loss = acc_ref[0] / jnp.maximum(acc_ref[1], 1.0)
        neg_loss = (acc_ref[3] - acc_ref[2]) / jnp.maximum(acc_ref[3], 1.0)
        out_ref[0] = pos_loss + neg_loss


@functools.partial(jax.jit, static_argnames=())
def kernel(embeddings, labels, embedding_memory, label_memory):
    out = pl.pallas_call(
        _loss_block,
        grid=(_NBLK,),
        in_specs=[
            pl.BlockSpec((_B, _D), lambda j: (0, 0)),
            pl.BlockSpec((_B,), lambda j: (0,)),
            pl.BlockSpec((_BLK, _D), lambda j: (j, 0)),
            pl.BlockSpec((_BLK,), lambda j: (j,)),
        ],
        out_specs=pl.BlockSpec(memory_space=pltpu.SMEM),
        out_shape=jax.ShapeDtypeStruct((1,), jnp.float32),
        scratch_shapes=[
            pltpu.VMEM((_B, _D + 2), jnp.float32),
            pltpu.VMEM((_B, 1), jnp.int32),
            pltpu.SMEM((4,), jnp.float32),
        ],
    )(embeddings, labels, embedding_memory, label_memory)
    return out[0]


# speedup vs baseline: 1.1420x; 1.1051x over previous
import functools

import jax
import jax.numpy as jnp
from jax.experimental import pallas as pl
from jax.experimental.pallas import tpu as pltpu

_B = 1024
_D = 128
_M = 16384
_BLK = 2048
_NBLK = _M // _BLK


def _loss_block(x_ref, lab_ref, mem_ref, lmem_ref, out_ref,
                xa_ref, lcol_ref, acc_ref):
    j = pl.program_id(0)
    is_batch = j == 0

    @pl.when(is_batch)
    def _init():
        acc_ref[0] = 0.0
        acc_ref[1] = 0.0
        acc_ref[2] = 0.0
        acc_ref[3] = 0.0
        x = x_ref[...]
        xsq = jnp.sum(x * x, axis=1, keepdims=True)
        xa_ref[...] = jnp.concatenate(
            [x * jnp.float32(-2.0), xsq, jnp.ones((_B, 1), jnp.float32)],
            axis=1)
        lcol_ref[...] = lab_ref[...].reshape(1, _B).T

    y0 = jnp.concatenate(
        [x_ref[...], mem_ref[pl.ds(_B, _BLK - _B), :]], axis=0)
    ly0 = jnp.concatenate(
        [lab_ref[...].reshape(1, _B),
         lmem_ref[pl.ds(_B, _BLK - _B)].reshape(1, _BLK - _B)], axis=1)
    y = jnp.where(is_batch, y0, mem_ref[...])
    ly = jnp.where(is_batch, ly0, lmem_ref[...].reshape(1, _BLK))

    ysq = jnp.sum(y * y, axis=1, keepdims=True)
    ya = jnp.concatenate(
        [y, jnp.ones((_BLK, 1), jnp.float32), ysq], axis=1)
    d2 = jax.lax.dot_general(
        xa_ref[...], ya, (((1,), (1,)), ((), ())),
        preferred_element_type=jnp.float32)
    m = jnp.maximum(d2, 1e-12)
    d = m * jax.lax.rsqrt(m)

    one = jnp.float32(1.0)
    zero = jnp.float32(0.0)
    matches = lcol_ref[...] == ly
    mf = jnp.where(matches, one, zero)
    ltf = jnp.where(m < one, one, zero)
    nf = jnp.where(matches, zero, ltf)

    acc_ref[0] = acc_ref[0] + jnp.sum(d * mf)
    acc_ref[1] = acc_ref[1] + jnp.sum(mf)
    acc_ref[2] = acc_ref[2] + jnp.sum(d * nf)
    acc_ref[3] = acc_ref[3] + jnp.sum(nf)

    @pl.when(is_batch)
    def _self_correction():
        rows = jax.lax.broadcasted_iota(jnp.int32, (_B, _BLK), 0)
        cols = jax.lax.broadcasted_iota(jnp.int32, (_B, _BLK), 1)
        diag_sum = jnp.sum(jnp.where(rows == cols, d, zero))
        acc_ref[0] = acc_ref[0] - diag_sum
        acc_ref[1] = acc_ref[1] - jnp.float32(_B)

    @pl.when(j == _NBLK - 1)
    def _finish():
        pos_loss = acc_ref[0] / jnp.maximum(acc_ref[1], 1.0)
        neg_loss = (acc_ref[3] - acc_ref[2]) / jnp.maximum(acc_ref[3], 1.0)
        out_ref[0] = pos_loss + neg_loss


@functools.partial(jax.jit, static_argnames=())
def kernel(embeddings, labels, embedding_memory, label_memory):
    out = pl.pallas_call(
        _loss_block,
        grid=(_NBLK,),
        in_specs=[
            pl.BlockSpec((_B, _D), lambda j: (0, 0)),
            pl.BlockSpec((_B,), lambda j: (0,)),
            pl.BlockSpec((_BLK, _D), lambda j: (j, 0)),
            pl.BlockSpec((_BLK,), lambda j: (j,)),
        ],
        out_specs=pl.BlockSpec(memory_space=pltpu.SMEM),
        out_shape=jax.ShapeDtypeStruct((1,), jnp.float32),
        scratch_shapes=[
            pltpu.VMEM((_B, _D + 2), jnp.float32),
            pltpu.VMEM((_B, 1), jnp.int32),
            pltpu.SMEM((4,), jnp.float32),
        ],
    )(embeddings, labels, embedding_memory, label_memory)
    return out[0]
